# baseline (speedup 1.0000x reference)
import functools

import jax
import jax.numpy as jnp
from jax import lax
from jax.experimental import pallas as pl
from jax.experimental.pallas import tpu as pltpu
from jax.experimental.pallas import tpu_sc as plsc

N = 10000
E = 320000
D = 128

NC = 2
NS = 16
NW = NC * NS
L = 16

CHUNK = 128
EPT = E // NS
SLOTS = EPT // CHUNK + 1
CAP = SLOTS * CHUNK
NH = N // NC
ACR = NH + 8
TRIPLES = (SLOTS - 1) // 3
RSTRIDE = 312
RPT = 320


def _mm_body(h_ref, w_ref, a2_ref, wh_ref, s_ref):
    wh = jnp.dot(h_ref[...], w_ref[...], preferred_element_type=jnp.float32)
    wh_ref[...] = wh
    s_ref[...] = jnp.dot(wh, a2_ref[...], preferred_element_type=jnp.float32)


def _stage1(h, W, A2):
    blk = 1000
    return pl.pallas_call(
        _mm_body,
        grid=(N // blk,),
        in_specs=[
            pl.BlockSpec((blk, D), lambda i: (i, 0)),
            pl.BlockSpec((D, D), lambda i: (0, 0)),
            pl.BlockSpec((D, D), lambda i: (0, 0)),
        ],
        out_specs=[
            pl.BlockSpec((blk, D), lambda i: (i, 0)),
            pl.BlockSpec((blk, D), lambda i: (i, 0)),
        ],
        out_shape=[
            jax.ShapeDtypeStruct((N, D), jnp.float32),
            jax.ShapeDtypeStruct((N, D), jnp.float32),
        ],
    )(h, W, A2)


def _part_body(rows_hbm, cols_hbm,
               rp_hbm, cp_hbm, cnt_hbm,
               in_r, in_c, lr0, lc0, lr1, lc1, cnt_v):
    c = lax.axis_index("c")
    s = lax.axis_index("s")

    @pl.when(c == 0)
    def _run():
        base = s * EPT
        pltpu.sync_copy(rows_hbm.at[pl.ds(base, EPT)], in_r)
        pltpu.sync_copy(cols_hbm.at[pl.ds(base, EPT)], in_c)

        def step(i, offs):
            o0, o1 = offs
            rv = in_r[pl.ds(i * L, L)]
            cv = in_c[pl.ds(i * L, L)]
            m0 = rv < NH
            m1 = jnp.logical_not(m0)
            plsc.store_compressed(lr0.at[pl.ds(o0, L)], rv, mask=m0)
            plsc.store_compressed(lc0.at[pl.ds(o0, L)], cv, mask=m0)
            plsc.store_compressed(lr1.at[pl.ds(o1, L)], rv, mask=m1)
            plsc.store_compressed(lc1.at[pl.ds(o1, L)], cv, mask=m1)
            n0 = jnp.max(plsc.all_reduce_population_count(m0))
            return (o0 + n0, o1 + (L - n0))

        off0, off1 = lax.fori_loop(0, EPT // L, step, (jnp.int32(0),
                                                       jnp.int32(0)))

        dumr = jnp.full((L,), N, jnp.int32)
        dumc = jnp.zeros((L,), jnp.int32)
        for k in range(CHUNK // L):
            lr0[pl.ds(off0 + k * L, L)] = dumr
            lc0[pl.ds(off0 + k * L, L)] = dumc
            lr1[pl.ds(off1 + k * L, L)] = dumr
            lc1[pl.ds(off1 + k * L, L)] = dumc

        nb0 = (off0 + CHUNK - 1) // CHUNK
        nb1 = (off1 + CHUNK - 1) // CHUNK

        lane = lax.iota(jnp.int32, L)
        cnt_v[...] = jnp.where(lane == 0, nb0, jnp.where(lane == 1, nb1, 0))
        pltpu.sync_copy(cnt_v, cnt_hbm.at[pl.ds(s * L, L)])

        pltpu.sync_copy(lr0.at[pl.ds(0, CAP)], rp_hbm.at[pl.ds(s * CAP, CAP)])
        pltpu.sync_copy(lc0.at[pl.ds(0, CAP)], cp_hbm.at[pl.ds(s * CAP, CAP)])
        pltpu.sync_copy(lr1.at[pl.ds(0, CAP)],
                        rp_hbm.at[pl.ds((NS + s) * CAP, CAP)])
        pltpu.sync_copy(lc1.at[pl.ds(0, CAP)],
                        cp_hbm.at[pl.ds((NS + s) * CAP, CAP)])


@functools.partial(
    pl.kernel,
    out_type=[
        jax.ShapeDtypeStruct((NC * NS * CAP,), jnp.int32),
        jax.ShapeDtypeStruct((NC * NS * CAP,), jnp.int32),
        jax.ShapeDtypeStruct((NS * L,), jnp.int32),
    ],
    mesh=plsc.VectorSubcoreMesh(core_axis_name="c", subcore_axis_name="s"),
    compiler_params=pltpu.CompilerParams(needs_layout_passes=False),
    scratch_types=[
        pltpu.VMEM((EPT,), jnp.int32),
        pltpu.VMEM((EPT,), jnp.int32),
        pltpu.VMEM((CAP + CHUNK,), jnp.int32),
        pltpu.VMEM((CAP + CHUNK,), jnp.int32),
        pltpu.VMEM((CAP + CHUNK,), jnp.int32),
        pltpu.VMEM((CAP + CHUNK,), jnp.int32),
        pltpu.VMEM((L,), jnp.int32),
    ],
)
def _stage15(rows_hbm, cols_hbm, rp_hbm, cp_hbm, cnt_hbm, *scratch):
    _part_body(rows_hbm, cols_hbm, rp_hbm, cp_hbm, cnt_hbm, *scratch)


def _sc_body(rp_hbm, cp_hbm, cnt_hbm, s1_hbm, s2_hbm, wh_hbm,
             psum_hbm, sump_hbm,
             s1_v, s2_v, cnt_v,
             r0, r1, r2, c0, c1, c2, x0, x1, x2, w0, w1, w2, f0, f1, f2,
             sumexp_v, accum,
             rs0, rs1, rs2, cs0, cs1, cs2, gs0, gs1, gs2, ss0, ss1, ss2):
    rows_vs = [r0, r1, r2]
    cols_vs = [c0, c1, c2]
    ridx_vs = [x0, x1, x2]
    w_vs = [w0, w1, w2]
    feat_vs = [f0, f1, f2]
    rsem = [rs0, rs1, rs2]
    csem = [cs0, cs1, cs2]
    gsem = [gs0, gs1, gs2]
    ssem = [ss0, ss1, ss2]

    c = lax.axis_index("c")
    s = lax.axis_index("s")
    wid = c * NS + s

    pltpu.sync_copy(s1_hbm, s1_v)
    pltpu.sync_copy(s2_hbm, s2_v)
    pltpu.sync_copy(cnt_hbm.at[pl.ds(s * L, L)], cnt_v)
    nb = jnp.max(plsc.load_gather(cnt_v, [jnp.full((L,), c, jnp.int32)]))

    zeros16 = jnp.zeros((L,), jnp.float32)

    def _z1(i, _):
        sumexp_v[pl.ds(i * L, L)] = zeros16
        return 0
    lax.fori_loop(0, (N + L) // L, _z1, 0)

    def _z2(i, _):
        for j in range(D // L):
            f0[i, pl.ds(j * L, L)] = zeros16
        return 0
    lax.fori_loop(0, CHUNK, _z2, 0)

    base_row = s * RSTRIDE
    for z in range(RPT // CHUNK):
        pltpu.sync_copy(f0, accum.at[pl.ds(base_row + z * CHUNK, CHUNK)])
    rem = RPT % CHUNK
    pltpu.sync_copy(f0.at[pl.ds(0, rem)],
                    accum.at[pl.ds(base_row + (RPT // CHUNK) * CHUNK, rem)])

    @pl.when(s == 0)
    def _zd():
        pltpu.sync_copy(f0.at[pl.ds(0, ACR - NH)],
                        accum.at[pl.ds(NH, ACR - NH)])

    plsc.subcore_barrier()

    rbase = (c * NS + s) * CAP

    def eoff(g):
        return rbase + g * CHUNK

    def issue_idx(g, b):
        pltpu.async_copy(rp_hbm.at[pl.ds(eoff(g), CHUNK)], rows_vs[b],
                         rsem[b])
        pltpu.async_copy(cp_hbm.at[pl.ds(eoff(g), CHUNK)], cols_vs[b],
                         csem[b])

    def wait_idx(g, b):
        pltpu.make_async_copy(rp_hbm.at[pl.ds(eoff(g), CHUNK)], rows_vs[b],
                              rsem[b]).wait()
        pltpu.make_async_copy(cp_hbm.at[pl.ds(eoff(g), CHUNK)], cols_vs[b],
                              csem[b]).wait()

    def issue_gather(b):
        pltpu.async_copy(wh_hbm.at[cols_vs[b]], feat_vs[b], gsem[b])

    def wait_gather(b):
        pltpu.make_async_copy(wh_hbm.at[cols_vs[b]], feat_vs[b],
                              gsem[b]).wait()

    def issue_scatter(b):
        pltpu.async_copy(feat_vs[b], accum.at[ridx_vs[b]], ssem[b], add=True)

    def wait_scatter(b):
        pltpu.make_async_copy(feat_vs[b], accum.at[ridx_vs[b]],
                              ssem[b]).wait()

    half_lo = c * NH

    def weights(b):
        for i in range(CHUNK // L):
            rv = rows_vs[b][pl.ds(i * L, L)]
            cv = cols_vs[b][pl.ds(i * L, L)]
            x = plsc.load_gather(s1_v, [rv]) + plsc.load_gather(s2_v, [cv])
            e = jnp.where(x > 0, x, 0.2 * x)
            w = jnp.exp(e)
            w_vs[b][pl.ds(i * L, L)] = w
            plsc.addupdate_scatter(sumexp_v, [rv], w)
            ri = rv - half_lo
            ri = jnp.where((ri >= 0) & (ri < NH), ri, NH)
            ridx_vs[b][pl.ds(i * L, L)] = ri

    def scale(b):
        wref = w_vs[b]
        fref = feat_vs[b]

        @plsc.parallel_loop(0, CHUNK, unroll=4)
        def _body(i):
            ws = plsc.load_gather(wref, [jnp.full((L,), i, jnp.int32)])
            for j in range(D // L):
                fref[i, pl.ds(j * L, L)] = fref[i, pl.ds(j * L, L)] * ws

    @pl.when(0 < nb)
    def _p0():
        issue_idx(0, 0)

    @pl.when(1 < nb)
    def _p1():
        issue_idx(1, 1)

    @pl.when(0 < nb)
    def _p2():
        wait_idx(0, 0)
        issue_gather(0)

    def triple(p, _):
        for j in range(3):
            g = 3 * p + j
            b = j
            nb1 = (j + 1) % 3
            nb2 = (j + 2) % 3

            @pl.when(g < nb)
            def _cmp():
                wait_gather(b)
                weights(b)
                scale(b)
                issue_scatter(b)

            @pl.when(g + 1 < nb)
            def _wi():
                wait_idx(g + 1, nb1)

            if j == 2:
                @pl.when(g - 2 < nb)
                def _ws():
                    wait_scatter(nb1)
            else:
                @pl.when((p > 0) & (g - 2 < nb))
                def _ws2():
                    wait_scatter(nb1)

            @pl.when(g + 1 < nb)
            def _ig():
                issue_gather(nb1)

            @pl.when(g + 2 < nb)
            def _ii():
                issue_idx(g + 2, nb2)
        return 0

    lax.fori_loop(0, TRIPLES, triple, 0)

    tg = SLOTS - 1

    @pl.when(tg < nb)
    def _tail():
        wait_gather(0)
        weights(0)
        scale(0)
        issue_scatter(0)

    for k in range(3):
        kk = k

        @pl.when((tg - 2 + kk >= 0) & (tg - 2 + kk < nb))
        def _dk():
            wait_scatter((1 + kk) % 3)

    plsc.subcore_barrier()

    pltpu.sync_copy(accum.at[pl.ds(base_row, RPT)],
                    psum_hbm.at[pl.ds(c * NH + base_row, RPT)])
    pltpu.sync_copy(sumexp_v.at[pl.ds(0, N)],
                    sump_hbm.at[pl.ds(wid * N, N)])


@functools.partial(
    pl.kernel,
    out_type=[
        jax.ShapeDtypeStruct((N, D), jnp.float32),
        jax.ShapeDtypeStruct((NW * N,), jnp.float32),
    ],
    mesh=plsc.VectorSubcoreMesh(core_axis_name="c", subcore_axis_name="s"),
    compiler_params=pltpu.CompilerParams(needs_layout_passes=False),
    scratch_types=[
        pltpu.VMEM((N,), jnp.float32),
        pltpu.VMEM((N,), jnp.float32),
        pltpu.VMEM((L,), jnp.int32),
        pltpu.VMEM((CHUNK,), jnp.int32),
        pltpu.VMEM((CHUNK,), jnp.int32),
        pltpu.VMEM((CHUNK,), jnp.int32),
        pltpu.VMEM((CHUNK,), jnp.int32),
        pltpu.VMEM((CHUNK,), jnp.int32),
        pltpu.VMEM((CHUNK,), jnp.int32),
        pltpu.VMEM((CHUNK,), jnp.int32),
        pltpu.VMEM((CHUNK,), jnp.int32),
        pltpu.VMEM((CHUNK,), jnp.int32),
        pltpu.VMEM((CHUNK,), jnp.float32),
        pltpu.VMEM((CHUNK,), jnp.float32),
        pltpu.VMEM((CHUNK,), jnp.float32),
        pltpu.VMEM((CHUNK, D), jnp.float32),
        pltpu.VMEM((CHUNK, D), jnp.float32),
        pltpu.VMEM((CHUNK, D), jnp.float32),
        pltpu.VMEM((N + L,), jnp.float32),
        pltpu.VMEM_SHARED((ACR, D), jnp.float32),
        pltpu.SemaphoreType.DMA,
        pltpu.SemaphoreType.DMA,
        pltpu.SemaphoreType.DMA,
        pltpu.SemaphoreType.DMA,
        pltpu.SemaphoreType.DMA,
        pltpu.SemaphoreType.DMA,
        pltpu.SemaphoreType.DMA,
        pltpu.SemaphoreType.DMA,
        pltpu.SemaphoreType.DMA,
        pltpu.SemaphoreType.DMA,
        pltpu.SemaphoreType.DMA,
        pltpu.SemaphoreType.DMA,
    ],
)
def _stage2(rp_hbm, cp_hbm, cnt_hbm, s1_hbm, s2_hbm, wh_hbm,
            psum_hbm, sump_hbm, *scratch):
    _sc_body(rp_hbm, cp_hbm, cnt_hbm, s1_hbm, s2_hbm, wh_hbm,
             psum_hbm, sump_hbm, *scratch)


def _fin_body(p_ref, sp_ref, o_ref):
    den = jnp.sum(sp_ref[0], axis=0) + 1e-10
    x = p_ref[...] / den[:, None]
    o_ref[...] = jnp.where(x > 0, x, jnp.exp(x) - 1.0)


def _stage3(psum, sumpT):
    blk = 1000
    return pl.pallas_call(
        _fin_body,
        grid=(N // blk,),
        in_specs=[
            pl.BlockSpec((blk, D), lambda i: (i, 0)),
            pl.BlockSpec((1, NW, blk), lambda i: (i, 0, 0)),
        ],
        out_specs=pl.BlockSpec((blk, D), lambda i: (i, 0)),
        out_shape=jax.ShapeDtypeStruct((N, D), jnp.float32),
    )(psum, sumpT)


def kernel(h, edge_index, W, a):
    rows = edge_index[0]
    cols = edge_index[1]
    a1 = a[:D, 0]
    a2 = a[D:, 0]
    A2 = jnp.zeros((D, D), jnp.float32).at[:, 0].set(a1).at[:, 1].set(a2)

    Wh, S = _stage1(h, W, A2)
    s1 = jnp.asarray(S[:, 0])
    s2 = jnp.asarray(S[:, 1])

    rp, cp, cnt = _stage15(rows, cols)
    psum, sump = _stage2(rp, cp, cnt, s1, s2, Wh)

    blk = 1000
    sumpT = sump.reshape(NW, N // blk, blk).transpose(1, 0, 2)
    return _stage3(psum, sumpT)


# speedup vs baseline: 1.0391x; 1.0391x over previous
import functools

import jax
import jax.numpy as jnp
from jax import lax
from jax.experimental import pallas as pl
from jax.experimental.pallas import tpu as pltpu
from jax.experimental.pallas import tpu_sc as plsc

N = 10000
E = 320000
D = 128

NC = 2
NS = 16
NW = NC * NS
L = 16

CHUNK = 128
EPT = E // NS
SLOTS = EPT // CHUNK + 1
CAP = SLOTS * CHUNK
NH = N // NC
ACR = NH + 8
TRIPLES = (SLOTS - 1) // 3
RSTRIDE = 312
RPT = 320


def _mm_body(h_ref, w_ref, a2_ref, wh_ref, s_ref):
    wh = jnp.dot(h_ref[...], w_ref[...], preferred_element_type=jnp.float32)
    wh_ref[...] = wh
    s_ref[...] = jnp.dot(wh, a2_ref[...], preferred_element_type=jnp.float32)


def _stage1(h, W, A2):
    blk = 1000
    return pl.pallas_call(
        _mm_body,
        grid=(N // blk,),
        in_specs=[
            pl.BlockSpec((blk, D), lambda i: (i, 0)),
            pl.BlockSpec((D, D), lambda i: (0, 0)),
            pl.BlockSpec((D, D), lambda i: (0, 0)),
        ],
        out_specs=[
            pl.BlockSpec((blk, D), lambda i: (i, 0)),
            pl.BlockSpec((blk, D), lambda i: (i, 0)),
        ],
        out_shape=[
            jax.ShapeDtypeStruct((N, D), jnp.float32),
            jax.ShapeDtypeStruct((N, D), jnp.float32),
        ],
    )(h, W, A2)


def _part_body(rows_hbm, cols_hbm,
               rp_hbm, cp_hbm, cnt_hbm,
               in_r, in_c, lr0, lc0, lr1, lc1, cnt_v):
    c = lax.axis_index("c")
    s = lax.axis_index("s")

    @pl.when(c == 0)
    def _run():
        base = s * EPT
        pltpu.sync_copy(rows_hbm.at[pl.ds(base, EPT)], in_r)
        pltpu.sync_copy(cols_hbm.at[pl.ds(base, EPT)], in_c)

        def step(i, offs):
            o0, o1 = offs
            rv = in_r[pl.ds(i * L, L)]
            cv = in_c[pl.ds(i * L, L)]
            m0 = rv < NH
            m1 = jnp.logical_not(m0)
            plsc.store_compressed(lr0.at[pl.ds(o0, L)], rv, mask=m0)
            plsc.store_compressed(lc0.at[pl.ds(o0, L)], cv, mask=m0)
            plsc.store_compressed(lr1.at[pl.ds(o1, L)], rv, mask=m1)
            plsc.store_compressed(lc1.at[pl.ds(o1, L)], cv, mask=m1)
            n0 = jnp.max(plsc.all_reduce_population_count(m0))
            return (o0 + n0, o1 + (L - n0))

        off0, off1 = lax.fori_loop(0, EPT // L, step, (jnp.int32(0),
                                                       jnp.int32(0)))

        dumr = jnp.full((L,), N, jnp.int32)
        dumc = jnp.zeros((L,), jnp.int32)
        for k in range(CHUNK // L):
            lr0[pl.ds(off0 + k * L, L)] = dumr
            lc0[pl.ds(off0 + k * L, L)] = dumc
            lr1[pl.ds(off1 + k * L, L)] = dumr
            lc1[pl.ds(off1 + k * L, L)] = dumc

        nb0 = (off0 + CHUNK - 1) // CHUNK
        nb1 = (off1 + CHUNK - 1) // CHUNK

        lane = lax.iota(jnp.int32, L)
        cnt_v[...] = jnp.where(lane == 0, nb0, jnp.where(lane == 1, nb1, 0))
        pltpu.sync_copy(cnt_v, cnt_hbm.at[pl.ds(s * L, L)])

        pltpu.sync_copy(lr0.at[pl.ds(0, CAP)], rp_hbm.at[pl.ds(s * CAP, CAP)])
        pltpu.sync_copy(lc0.at[pl.ds(0, CAP)], cp_hbm.at[pl.ds(s * CAP, CAP)])
        pltpu.sync_copy(lr1.at[pl.ds(0, CAP)],
                        rp_hbm.at[pl.ds((NS + s) * CAP, CAP)])
        pltpu.sync_copy(lc1.at[pl.ds(0, CAP)],
                        cp_hbm.at[pl.ds((NS + s) * CAP, CAP)])


@functools.partial(
    pl.kernel,
    out_type=[
        jax.ShapeDtypeStruct((NC * NS * CAP,), jnp.int32),
        jax.ShapeDtypeStruct((NC * NS * CAP,), jnp.int32),
        jax.ShapeDtypeStruct((NS * L,), jnp.int32),
    ],
    mesh=plsc.VectorSubcoreMesh(core_axis_name="c", subcore_axis_name="s"),
    compiler_params=pltpu.CompilerParams(needs_layout_passes=False),
    scratch_types=[
        pltpu.VMEM((EPT,), jnp.int32),
        pltpu.VMEM((EPT,), jnp.int32),
        pltpu.VMEM((CAP + CHUNK,), jnp.int32),
        pltpu.VMEM((CAP + CHUNK,), jnp.int32),
        pltpu.VMEM((CAP + CHUNK,), jnp.int32),
        pltpu.VMEM((CAP + CHUNK,), jnp.int32),
        pltpu.VMEM((L,), jnp.int32),
    ],
)
def _stage15(rows_hbm, cols_hbm, rp_hbm, cp_hbm, cnt_hbm, *scratch):
    _part_body(rows_hbm, cols_hbm, rp_hbm, cp_hbm, cnt_hbm, *scratch)


def _sc_body(rp_hbm, cp_hbm, cnt_hbm, s1_hbm, s2_hbm, wh_hbm,
             psum_hbm, sump_hbm,
             s1_v, s2_v, cnt_v,
             r0, r1, r2, c0, c1, c2, x0, x1, x2, w0, w1, w2, f0, f1, f2,
             sumexp_v, accum,
             rs0, rs1, rs2, cs0, cs1, cs2, gs0, gs1, gs2, ss0, ss1, ss2):
    rows_vs = [r0, r1, r2]
    cols_vs = [c0, c1, c2]
    ridx_vs = [x0, x1, x2]
    w_vs = [w0, w1, w2]
    feat_vs = [f0, f1, f2]
    rsem = [rs0, rs1, rs2]
    csem = [cs0, cs1, cs2]
    gsem = [gs0, gs1, gs2]
    ssem = [ss0, ss1, ss2]

    c = lax.axis_index("c")
    s = lax.axis_index("s")
    wid = c * NS + s

    pltpu.sync_copy(s1_hbm, s1_v)
    pltpu.sync_copy(s2_hbm, s2_v)
    pltpu.sync_copy(cnt_hbm.at[pl.ds(s * L, L)], cnt_v)
    nb = jnp.max(plsc.load_gather(cnt_v, [jnp.full((L,), c, jnp.int32)]))

    zeros16 = jnp.zeros((L,), jnp.float32)

    def _z1(i, _):
        sumexp_v[pl.ds(i * L, L)] = zeros16
        return 0
    lax.fori_loop(0, (N + L) // L, _z1, 0)

    def _z2(i, _):
        for j in range(D // L):
            f0[i, pl.ds(j * L, L)] = zeros16
        return 0
    lax.fori_loop(0, CHUNK, _z2, 0)

    base_row = s * RSTRIDE
    for z in range(RPT // CHUNK):
        pltpu.sync_copy(f0, accum.at[pl.ds(base_row + z * CHUNK, CHUNK)])
    rem = RPT % CHUNK
    pltpu.sync_copy(f0.at[pl.ds(0, rem)],
                    accum.at[pl.ds(base_row + (RPT // CHUNK) * CHUNK, rem)])

    @pl.when(s == 0)
    def _zd():
        pltpu.sync_copy(f0.at[pl.ds(0, ACR - NH)],
                        accum.at[pl.ds(NH, ACR - NH)])

    plsc.subcore_barrier()

    rbase = (c * NS + s) * CAP

    def eoff(g):
        return rbase + g * CHUNK

    def issue_idx(g, b):
        pltpu.async_copy(rp_hbm.at[pl.ds(eoff(g), CHUNK)], rows_vs[b],
                         rsem[b])
        pltpu.async_copy(cp_hbm.at[pl.ds(eoff(g), CHUNK)], cols_vs[b],
                         csem[b])

    def wait_idx(g, b):
        pltpu.make_async_copy(rp_hbm.at[pl.ds(eoff(g), CHUNK)], rows_vs[b],
                              rsem[b]).wait()
        pltpu.make_async_copy(cp_hbm.at[pl.ds(eoff(g), CHUNK)], cols_vs[b],
                              csem[b]).wait()

    def issue_gather(b):
        pltpu.async_copy(wh_hbm.at[cols_vs[b]], feat_vs[b], gsem[b])

    def wait_gather(b):
        pltpu.make_async_copy(wh_hbm.at[cols_vs[b]], feat_vs[b],
                              gsem[b]).wait()

    def issue_scatter(b):
        pltpu.async_copy(feat_vs[b], accum.at[ridx_vs[b]], ssem[b], add=True)

    def wait_scatter(b):
        pltpu.make_async_copy(feat_vs[b], accum.at[ridx_vs[b]],
                              ssem[b]).wait()

    half_lo = c * NH

    def weights(b):
        for i in range(CHUNK // L):
            rv = rows_vs[b][pl.ds(i * L, L)]
            cv = cols_vs[b][pl.ds(i * L, L)]
            x = plsc.load_gather(s1_v, [rv]) + plsc.load_gather(s2_v, [cv])
            e = jnp.where(x > 0, x, 0.2 * x)
            w = jnp.exp(e)
            w_vs[b][pl.ds(i * L, L)] = w
            plsc.addupdate_scatter(sumexp_v, [rv], w)
            ri = rv - half_lo
            ri = jnp.where((ri >= 0) & (ri < NH), ri, NH)
            ridx_vs[b][pl.ds(i * L, L)] = ri

    def scale(b):
        wref = w_vs[b]
        fref = feat_vs[b]

        @plsc.parallel_loop(0, CHUNK, unroll=4)
        def _body(i):
            ws = plsc.load_gather(wref, [jnp.full((L,), i, jnp.int32)])
            for j in range(D // L):
                fref[i, pl.ds(j * L, L)] = fref[i, pl.ds(j * L, L)] * ws

    @pl.when(nb > 0)
    def _prime():
        issue_idx(0, 0)

    def triple(p, _):
        for j in range(3):
            g = 3 * p + j
            b = j
            nbuf = (j + 1) % 3

            @pl.when(g < nb)
            def _ab():
                wait_idx(g, b)
                issue_gather(b)

            if j < 2:
                @pl.when((p > 0) & (g - 2 < nb))
                def _w():
                    wait_scatter(nbuf)
            else:
                @pl.when(g - 2 < nb)
                def _w2():
                    wait_scatter(nbuf)

            @pl.when(g + 1 < nb)
            def _i():
                issue_idx(g + 1, nbuf)

            @pl.when(g < nb)
            def _cmp():
                weights(b)
                wait_gather(b)
                scale(b)
                issue_scatter(b)
        return 0

    lax.fori_loop(0, TRIPLES, triple, 0)

    tg = SLOTS - 1

    @pl.when(tg < nb)
    def _tail():
        wait_idx(tg, 0)
        issue_gather(0)
        weights(0)
        wait_gather(0)
        scale(0)
        issue_scatter(0)

    for k in range(3):
        kk = k

        @pl.when((tg - 2 + kk >= 0) & (tg - 2 + kk < nb))
        def _dk():
            wait_scatter((1 + kk) % 3)

    plsc.subcore_barrier()

    pltpu.sync_copy(accum.at[pl.ds(base_row, RPT)],
                    psum_hbm.at[pl.ds(c * NH + base_row, RPT)])
    pltpu.sync_copy(sumexp_v.at[pl.ds(0, N)],
                    sump_hbm.at[pl.ds(wid * N, N)])


@functools.partial(
    pl.kernel,
    out_type=[
        jax.ShapeDtypeStruct((N, D), jnp.float32),
        jax.ShapeDtypeStruct((NW * N,), jnp.float32),
    ],
    mesh=plsc.VectorSubcoreMesh(core_axis_name="c", subcore_axis_name="s"),
    compiler_params=pltpu.CompilerParams(needs_layout_passes=False),
    scratch_types=[
        pltpu.VMEM((N,), jnp.float32),
        pltpu.VMEM((N,), jnp.float32),
        pltpu.VMEM((L,), jnp.int32),
        pltpu.VMEM((CHUNK,), jnp.int32),
        pltpu.VMEM((CHUNK,), jnp.int32),
        pltpu.VMEM((CHUNK,), jnp.int32),
        pltpu.VMEM((CHUNK,), jnp.int32),
        pltpu.VMEM((CHUNK,), jnp.int32),
        pltpu.VMEM((CHUNK,), jnp.int32),
        pltpu.VMEM((CHUNK,), jnp.int32),
        pltpu.VMEM((CHUNK,), jnp.int32),
        pltpu.VMEM((CHUNK,), jnp.int32),
        pltpu.VMEM((CHUNK,), jnp.float32),
        pltpu.VMEM((CHUNK,), jnp.float32),
        pltpu.VMEM((CHUNK,), jnp.float32),
        pltpu.VMEM((CHUNK, D), jnp.float32),
        pltpu.VMEM((CHUNK, D), jnp.float32),
        pltpu.VMEM((CHUNK, D), jnp.float32),
        pltpu.VMEM((N + L,), jnp.float32),
        pltpu.VMEM_SHARED((ACR, D), jnp.float32),
        pltpu.SemaphoreType.DMA,
        pltpu.SemaphoreType.DMA,
        pltpu.SemaphoreType.DMA,
        pltpu.SemaphoreType.DMA,
        pltpu.SemaphoreType.DMA,
        pltpu.SemaphoreType.DMA,
        pltpu.SemaphoreType.DMA,
        pltpu.SemaphoreType.DMA,
        pltpu.SemaphoreType.DMA,
        pltpu.SemaphoreType.DMA,
        pltpu.SemaphoreType.DMA,
        pltpu.SemaphoreType.DMA,
    ],
)
def _stage2(rp_hbm, cp_hbm, cnt_hbm, s1_hbm, s2_hbm, wh_hbm,
            psum_hbm, sump_hbm, *scratch):
    _sc_body(rp_hbm, cp_hbm, cnt_hbm, s1_hbm, s2_hbm, wh_hbm,
             psum_hbm, sump_hbm, *scratch)


def _fin_body(p_ref, sp_ref, o_ref):
    den = jnp.sum(sp_ref[0], axis=0) + 1e-10
    x = p_ref[...] / den[:, None]
    o_ref[...] = jnp.where(x > 0, x, jnp.exp(x) - 1.0)


def _stage3(psum, sumpT):
    blk = 1000
    return pl.pallas_call(
        _fin_body,
        grid=(N // blk,),
        in_specs=[
            pl.BlockSpec((blk, D), lambda i: (i, 0)),
            pl.BlockSpec((1, NW, blk), lambda i: (i, 0, 0)),
        ],
        out_specs=pl.BlockSpec((blk, D), lambda i: (i, 0)),
        out_shape=jax.ShapeDtypeStruct((N, D), jnp.float32),
    )(psum, sumpT)


def kernel(h, edge_index, W, a):
    rows = edge_index[0]
    cols = edge_index[1]
    a1 = a[:D, 0]
    a2 = a[D:, 0]
    A2 = jnp.zeros((D, D), jnp.float32).at[:, 0].set(a1).at[:, 1].set(a2)

    Wh, S = _stage1(h, W, A2)
    s1 = jnp.asarray(S[:, 0])
    s2 = jnp.asarray(S[:, 1])

    rp, cp, cnt = _stage15(rows, cols)
    psum, sump = _stage2(rp, cp, cnt, s1, s2, Wh)

    blk = 1000
    sumpT = sump.reshape(NW, N // blk, blk).transpose(1, 0, 2)
    return _stage3(psum, sumpT)


# speedup vs baseline: 1.0459x; 1.0066x over previous
import functools

import jax
import jax.numpy as jnp
from jax import lax
from jax.experimental import pallas as pl
from jax.experimental.pallas import tpu as pltpu
from jax.experimental.pallas import tpu_sc as plsc

N = 10000
E = 320000
D = 128

NC = 2
NS = 16
NW = NC * NS
L = 16

CHUNK = 128
EPT = E // NS
SLOTS = EPT // CHUNK + 1
CAP = SLOTS * CHUNK
NH = N // NC
ACR = NH + 8
TRIPLES = (SLOTS - 1) // 3
RSTRIDE = 312
RPT = 320


def _mm_body(h_ref, w_ref, a2_ref, wh_ref, s_ref):
    wh = jnp.dot(h_ref[...], w_ref[...], preferred_element_type=jnp.float32)
    wh_ref[...] = wh
    s_ref[...] = jnp.dot(wh, a2_ref[...], preferred_element_type=jnp.float32)


def _stage1(h, W, A2):
    blk = 1000
    return pl.pallas_call(
        _mm_body,
        grid=(N // blk,),
        in_specs=[
            pl.BlockSpec((blk, D), lambda i: (i, 0)),
            pl.BlockSpec((D, D), lambda i: (0, 0)),
            pl.BlockSpec((D, D), lambda i: (0, 0)),
        ],
        out_specs=[
            pl.BlockSpec((blk, D), lambda i: (i, 0)),
            pl.BlockSpec((blk, D), lambda i: (i, 0)),
        ],
        out_shape=[
            jax.ShapeDtypeStruct((N, D), jnp.float32),
            jax.ShapeDtypeStruct((N, D), jnp.float32),
        ],
    )(h, W, A2)


def _part_body(rows_hbm, cols_hbm,
               rp_hbm, cp_hbm, cnt_hbm,
               in_r, in_c, lr0, lc0, lr1, lc1, cnt_v):
    c = lax.axis_index("c")
    s = lax.axis_index("s")

    @pl.when(c == 0)
    def _run():
        base = s * EPT
        pltpu.sync_copy(rows_hbm.at[pl.ds(base, EPT)], in_r)
        pltpu.sync_copy(cols_hbm.at[pl.ds(base, EPT)], in_c)

        def step(i, offs):
            o0, o1 = offs
            rv = in_r[pl.ds(i * L, L)]
            cv = in_c[pl.ds(i * L, L)]
            m0 = rv < NH
            m1 = jnp.logical_not(m0)
            plsc.store_compressed(lr0.at[pl.ds(o0, L)], rv, mask=m0)
            plsc.store_compressed(lc0.at[pl.ds(o0, L)], cv, mask=m0)
            plsc.store_compressed(lr1.at[pl.ds(o1, L)], rv, mask=m1)
            plsc.store_compressed(lc1.at[pl.ds(o1, L)], cv, mask=m1)
            n0 = jnp.max(plsc.all_reduce_population_count(m0))
            return (o0 + n0, o1 + (L - n0))

        off0, off1 = lax.fori_loop(0, EPT // L, step, (jnp.int32(0),
                                                       jnp.int32(0)))

        dumr = jnp.full((L,), N, jnp.int32)
        dumc = jnp.zeros((L,), jnp.int32)
        for k in range(CHUNK // L):
            lr0[pl.ds(off0 + k * L, L)] = dumr
            lc0[pl.ds(off0 + k * L, L)] = dumc
            lr1[pl.ds(off1 + k * L, L)] = dumr
            lc1[pl.ds(off1 + k * L, L)] = dumc

        nb0 = (off0 + CHUNK - 1) // CHUNK
        nb1 = (off1 + CHUNK - 1) // CHUNK

        lane = lax.iota(jnp.int32, L)
        cnt_v[...] = jnp.where(lane == 0, nb0, jnp.where(lane == 1, nb1, 0))
        pltpu.sync_copy(cnt_v, cnt_hbm.at[pl.ds(s * L, L)])

        pltpu.sync_copy(lr0.at[pl.ds(0, CAP)], rp_hbm.at[pl.ds(s * CAP, CAP)])
        pltpu.sync_copy(lc0.at[pl.ds(0, CAP)], cp_hbm.at[pl.ds(s * CAP, CAP)])
        pltpu.sync_copy(lr1.at[pl.ds(0, CAP)],
                        rp_hbm.at[pl.ds((NS + s) * CAP, CAP)])
        pltpu.sync_copy(lc1.at[pl.ds(0, CAP)],
                        cp_hbm.at[pl.ds((NS + s) * CAP, CAP)])


@functools.partial(
    pl.kernel,
    out_type=[
        jax.ShapeDtypeStruct((NC * NS * CAP,), jnp.int32),
        jax.ShapeDtypeStruct((NC * NS * CAP,), jnp.int32),
        jax.ShapeDtypeStruct((NS * L,), jnp.int32),
    ],
    mesh=plsc.VectorSubcoreMesh(core_axis_name="c", subcore_axis_name="s"),
    compiler_params=pltpu.CompilerParams(needs_layout_passes=False),
    scratch_types=[
        pltpu.VMEM((EPT,), jnp.int32),
        pltpu.VMEM((EPT,), jnp.int32),
        pltpu.VMEM((CAP + CHUNK,), jnp.int32),
        pltpu.VMEM((CAP + CHUNK,), jnp.int32),
        pltpu.VMEM((CAP + CHUNK,), jnp.int32),
        pltpu.VMEM((CAP + CHUNK,), jnp.int32),
        pltpu.VMEM((L,), jnp.int32),
    ],
)
def _stage15(rows_hbm, cols_hbm, rp_hbm, cp_hbm, cnt_hbm, *scratch):
    _part_body(rows_hbm, cols_hbm, rp_hbm, cp_hbm, cnt_hbm, *scratch)


def _sc_body(rp_hbm, cp_hbm, cnt_hbm, s1_hbm, s2_hbm, wh_hbm,
             psum_hbm, sump_hbm,
             s1_v, s2_v, cnt_v,
             r0, r1, r2, c0, c1, c2, x0, x1, x2, w0, w1, w2, f0, f1, f2,
             sumexp_v, accum,
             rs0, rs1, rs2, cs0, cs1, cs2, gs0, gs1, gs2, ss0, ss1, ss2):
    rows_vs = [r0, r1, r2]
    cols_vs = [c0, c1, c2]
    ridx_vs = [x0, x1, x2]
    w_vs = [w0, w1, w2]
    feat_vs = [f0, f1, f2]
    rsem = [rs0, rs1, rs2]
    csem = [cs0, cs1, cs2]
    gsem = [gs0, gs1, gs2]
    ssem = [ss0, ss1, ss2]

    c = lax.axis_index("c")
    s = lax.axis_index("s")
    wid = c * NS + s

    pltpu.sync_copy(s1_hbm, s1_v)
    pltpu.sync_copy(s2_hbm, s2_v)
    pltpu.sync_copy(cnt_hbm.at[pl.ds(s * L, L)], cnt_v)
    nb = jnp.max(plsc.load_gather(cnt_v, [jnp.full((L,), c, jnp.int32)]))

    zeros16 = jnp.zeros((L,), jnp.float32)

    def _z1(i, _):
        sumexp_v[pl.ds(i * L, L)] = zeros16
        return 0
    lax.fori_loop(0, (N + L) // L, _z1, 0)

    def _z2(i, _):
        for j in range(D // L):
            f0[i, pl.ds(j * L, L)] = zeros16
        return 0
    lax.fori_loop(0, CHUNK, _z2, 0)

    base_row = s * RSTRIDE
    for z in range(RPT // CHUNK):
        pltpu.sync_copy(f0, accum.at[pl.ds(base_row + z * CHUNK, CHUNK)])
    rem = RPT % CHUNK
    pltpu.sync_copy(f0.at[pl.ds(0, rem)],
                    accum.at[pl.ds(base_row + (RPT // CHUNK) * CHUNK, rem)])

    @pl.when(s == 0)
    def _zd():
        pltpu.sync_copy(f0.at[pl.ds(0, ACR - NH)],
                        accum.at[pl.ds(NH, ACR - NH)])

    plsc.subcore_barrier()

    rbase = (c * NS + s) * CAP

    def eoff(g):
        return rbase + g * CHUNK

    def issue_idx(g, b):
        pltpu.async_copy(rp_hbm.at[pl.ds(eoff(g), CHUNK)], rows_vs[b],
                         rsem[b])
        pltpu.async_copy(cp_hbm.at[pl.ds(eoff(g), CHUNK)], cols_vs[b],
                         csem[b])

    def wait_idx(g, b):
        pltpu.make_async_copy(rp_hbm.at[pl.ds(eoff(g), CHUNK)], rows_vs[b],
                              rsem[b]).wait()
        pltpu.make_async_copy(cp_hbm.at[pl.ds(eoff(g), CHUNK)], cols_vs[b],
                              csem[b]).wait()

    def issue_gather(b):
        pltpu.async_copy(wh_hbm.at[cols_vs[b]], feat_vs[b], gsem[b])

    def wait_gather(b):
        pltpu.make_async_copy(wh_hbm.at[cols_vs[b]], feat_vs[b],
                              gsem[b]).wait()

    def issue_scatter(b):
        pltpu.async_copy(feat_vs[b], accum.at[ridx_vs[b]], ssem[b], add=True)

    def wait_scatter(b):
        pltpu.make_async_copy(feat_vs[b], accum.at[ridx_vs[b]],
                              ssem[b]).wait()

    half_lo = c * NH

    def weights(b):
        for i in range(CHUNK // L):
            rv = rows_vs[b][pl.ds(i * L, L)]
            cv = cols_vs[b][pl.ds(i * L, L)]
            x = plsc.load_gather(s1_v, [rv]) + plsc.load_gather(s2_v, [cv])
            e = jnp.where(x > 0, x, 0.2 * x)
            w = jnp.exp(e)
            w_vs[b][pl.ds(i * L, L)] = w
            plsc.addupdate_scatter(sumexp_v, [rv], w)
            ri = rv - half_lo
            ri = jnp.where((ri >= 0) & (ri < NH), ri, NH)
            ridx_vs[b][pl.ds(i * L, L)] = ri

    def scale(b):
        wref = w_vs[b]
        fref = feat_vs[b]

        @plsc.parallel_loop(0, CHUNK, unroll=8)
        def _body(i):
            ws = plsc.load_gather(wref, [jnp.full((L,), i, jnp.int32)])
            for j in range(D // L):
                fref[i, pl.ds(j * L, L)] = fref[i, pl.ds(j * L, L)] * ws

    @pl.when(nb > 0)
    def _prime():
        issue_idx(0, 0)

    def triple(p, _):
        for j in range(3):
            g = 3 * p + j
            b = j
            nbuf = (j + 1) % 3

            @pl.when(g < nb)
            def _ab():
                wait_idx(g, b)
                issue_gather(b)

            if j < 2:
                @pl.when((p > 0) & (g - 2 < nb))
                def _w():
                    wait_scatter(nbuf)
            else:
                @pl.when(g - 2 < nb)
                def _w2():
                    wait_scatter(nbuf)

            @pl.when(g + 1 < nb)
            def _i():
                issue_idx(g + 1, nbuf)

            @pl.when(g < nb)
            def _cmp():
                weights(b)
                wait_gather(b)
                scale(b)
                issue_scatter(b)
        return 0

    lax.fori_loop(0, TRIPLES, triple, 0)

    tg = SLOTS - 1

    @pl.when(tg < nb)
    def _tail():
        wait_idx(tg, 0)
        issue_gather(0)
        weights(0)
        wait_gather(0)
        scale(0)
        issue_scatter(0)

    for k in range(3):
        kk = k

        @pl.when((tg - 2 + kk >= 0) & (tg - 2 + kk < nb))
        def _dk():
            wait_scatter((1 + kk) % 3)

    plsc.subcore_barrier()

    pltpu.sync_copy(accum.at[pl.ds(base_row, RPT)],
                    psum_hbm.at[pl.ds(c * NH + base_row, RPT)])
    pltpu.sync_copy(sumexp_v.at[pl.ds(0, N)],
                    sump_hbm.at[pl.ds(wid * N, N)])


@functools.partial(
    pl.kernel,
    out_type=[
        jax.ShapeDtypeStruct((N, D), jnp.float32),
        jax.ShapeDtypeStruct((NW * N,), jnp.float32),
    ],
    mesh=plsc.VectorSubcoreMesh(core_axis_name="c", subcore_axis_name="s"),
    compiler_params=pltpu.CompilerParams(needs_layout_passes=False),
    scratch_types=[
        pltpu.VMEM((N,), jnp.float32),
        pltpu.VMEM((N,), jnp.float32),
        pltpu.VMEM((L,), jnp.int32),
        pltpu.VMEM((CHUNK,), jnp.int32),
        pltpu.VMEM((CHUNK,), jnp.int32),
        pltpu.VMEM((CHUNK,), jnp.int32),
        pltpu.VMEM((CHUNK,), jnp.int32),
        pltpu.VMEM((CHUNK,), jnp.int32),
        pltpu.VMEM((CHUNK,), jnp.int32),
        pltpu.VMEM((CHUNK,), jnp.int32),
        pltpu.VMEM((CHUNK,), jnp.int32),
        pltpu.VMEM((CHUNK,), jnp.int32),
        pltpu.VMEM((CHUNK,), jnp.float32),
        pltpu.VMEM((CHUNK,), jnp.float32),
        pltpu.VMEM((CHUNK,), jnp.float32),
        pltpu.VMEM((CHUNK, D), jnp.float32),
        pltpu.VMEM((CHUNK, D), jnp.float32),
        pltpu.VMEM((CHUNK, D), jnp.float32),
        pltpu.VMEM((N + L,), jnp.float32),
        pltpu.VMEM_SHARED((ACR, D), jnp.float32),
        pltpu.SemaphoreType.DMA,
        pltpu.SemaphoreType.DMA,
        pltpu.SemaphoreType.DMA,
        pltpu.SemaphoreType.DMA,
        pltpu.SemaphoreType.DMA,
        pltpu.SemaphoreType.DMA,
        pltpu.SemaphoreType.DMA,
        pltpu.SemaphoreType.DMA,
        pltpu.SemaphoreType.DMA,
        pltpu.SemaphoreType.DMA,
        pltpu.SemaphoreType.DMA,
        pltpu.SemaphoreType.DMA,
    ],
)
def _stage2(rp_hbm, cp_hbm, cnt_hbm, s1_hbm, s2_hbm, wh_hbm,
            psum_hbm, sump_hbm, *scratch):
    _sc_body(rp_hbm, cp_hbm, cnt_hbm, s1_hbm, s2_hbm, wh_hbm,
             psum_hbm, sump_hbm, *scratch)


def _fin_body(p_ref, sp_ref, o_ref):
    den = jnp.sum(sp_ref[0], axis=0) + 1e-10
    x = p_ref[...] / den[:, None]
    o_ref[...] = jnp.where(x > 0, x, jnp.exp(x) - 1.0)


def _stage3(psum, sumpT):
    blk = 1000
    return pl.pallas_call(
        _fin_body,
        grid=(N // blk,),
        in_specs=[
            pl.BlockSpec((blk, D), lambda i: (i, 0)),
            pl.BlockSpec((1, NW, blk), lambda i: (i, 0, 0)),
        ],
        out_specs=pl.BlockSpec((blk, D), lambda i: (i, 0)),
        out_shape=jax.ShapeDtypeStruct((N, D), jnp.float32),
    )(psum, sumpT)


def kernel(h, edge_index, W, a):
    rows = edge_index[0]
    cols = edge_index[1]
    a1 = a[:D, 0]
    a2 = a[D:, 0]
    A2 = jnp.zeros((D, D), jnp.float32).at[:, 0].set(a1).at[:, 1].set(a2)

    Wh, S = _stage1(h, W, A2)
    s1 = jnp.asarray(S[:, 0])
    s2 = jnp.asarray(S[:, 1])

    rp, cp, cnt = _stage15(rows, cols)
    psum, sump = _stage2(rp, cp, cnt, s1, s2, Wh)

    blk = 1000
    sumpT = sump.reshape(NW, N // blk, blk).transpose(1, 0, 2)
    return _stage3(psum, sumpT)
